# baseline (speedup 1.0000x reference)
import functools
import math

import jax
import jax.numpy as jnp
from jax import lax
from jax.experimental import pallas as pl
from jax.experimental.pallas import tpu as pltpu
from jax.experimental.pallas import tpu_sc as plsc

D_MODEL = 64
SCALE = math.sqrt(D_MODEL)


@functools.lru_cache(maxsize=None)
def _build(B, D, interpret):
    NC, NS = 2, 16
    NW = NC * NS
    assert B % NW == 0
    b_per_w = B // NW
    CHUNK = 512
    assert b_per_w % CHUNK == 0
    n_chunks = b_per_w // CHUNK

    mesh = plsc.VectorSubcoreMesh(
        core_axis_name="c", subcore_axis_name="s", num_cores=NC, num_subcores=NS
    )

    @functools.partial(
        pl.kernel,
        mesh=mesh,
        out_type=jax.ShapeDtypeStruct((B, D), jnp.float32),
        scratch_types=[
            pltpu.VMEM((CHUNK,), jnp.int32),
            pltpu.VMEM((CHUNK, D), jnp.float32),
            pltpu.SemaphoreType.DMA,
        ],
        interpret=interpret,
        compiler_params=pltpu.CompilerParams(use_tc_tiling_on_sc=False),
    )
    def emb_kernel(idx_hbm, table_hbm, out_hbm, idx_v, rows_v, sem):
        wid = lax.axis_index("s") * NC + lax.axis_index("c")
        base = wid * b_per_w

        @pl.loop(0, n_chunks)
        def _chunk(ci):
            off = base + ci * CHUNK
            pltpu.sync_copy(idx_hbm.at[pl.ds(off, CHUNK)], idx_v)
            pltpu.async_copy(table_hbm.at[idx_v], rows_v, sem).wait()

            @pl.loop(0, CHUNK)
            def _scale(r):
                for j in range(D // 16):
                    sl = pl.ds(j * 16, 16)
                    rows_v[r, sl] = rows_v[r, sl] * SCALE

            pltpu.sync_copy(rows_v, out_hbm.at[pl.ds(off, CHUNK)])

    return emb_kernel


def kernel(x, table):
    B = x.shape[0] * x.shape[1]
    idx = x.reshape(B).astype(jnp.int32)
    out = _build(B, table.shape[1], False)(idx, table)
    return out.reshape(x.shape[0], x.shape[1], table.shape[1])


# speedup vs baseline: 1.0031x; 1.0031x over previous
import functools
import math

import jax
import jax.numpy as jnp
from jax import lax
from jax.experimental import pallas as pl
from jax.experimental.pallas import tpu as pltpu
from jax.experimental.pallas import tpu_sc as plsc

D_MODEL = 64
SCALE = math.sqrt(D_MODEL)
NBUF = 2


@functools.lru_cache(maxsize=None)
def _build(B, D, interpret):
    NC, NS = 2, 16
    NW = NC * NS
    assert B % NW == 0
    b_per_w = B // NW
    CHUNK = 320
    assert b_per_w % CHUNK == 0
    n_chunks = b_per_w // CHUNK
    assert n_chunks >= 2 * NBUF and (n_chunks - 2 * NBUF) % NBUF == 0

    mesh = plsc.VectorSubcoreMesh(
        core_axis_name="c", subcore_axis_name="s", num_cores=NC, num_subcores=NS
    )

    @functools.partial(
        pl.kernel,
        mesh=mesh,
        out_type=jax.ShapeDtypeStruct((B, D), jnp.float32),
        scratch_types=[
            pltpu.VMEM((b_per_w,), jnp.int32),
            pltpu.VMEM((NBUF, CHUNK, D), jnp.float32),
            pltpu.VMEM((NBUF, CHUNK, D), jnp.float32),
            pltpu.SemaphoreType.DMA((NBUF,)),
            pltpu.SemaphoreType.DMA((NBUF,)),
        ],
        interpret=interpret,
        compiler_params=pltpu.CompilerParams(use_tc_tiling_on_sc=False),
    )
    def emb_kernel(idx_hbm, table_hbm, out_hbm, idx_v, gbuf, sbuf, gsem, ssem):
        wid = lax.axis_index("s") * NC + lax.axis_index("c")
        base = wid * b_per_w
        pltpu.sync_copy(idx_hbm.at[pl.ds(base, b_per_w)], idx_v)

        def gather_start(g, b):
            pltpu.async_copy(
                table_hbm.at[idx_v.at[pl.ds(g * CHUNK, CHUNK)]],
                gbuf.at[b],
                gsem.at[b],
            )

        def gather_wait(b):
            pltpu.make_async_copy(
                table_hbm.at[idx_v.at[pl.ds(0, CHUNK)]], gbuf.at[b], gsem.at[b]
            ).wait()

        def scale(b):
            @pl.loop(0, CHUNK, unroll=4)
            def _row(r):
                for j in range(D // 16):
                    sl = pl.ds(j * 16, 16)
                    sbuf[b, r, sl] = gbuf[b, r, sl] * SCALE

        def store_start(g, b):
            pltpu.async_copy(
                sbuf.at[b],
                out_hbm.at[pl.ds(base + g * CHUNK, CHUNK)],
                ssem.at[b],
            )

        def store_wait(b):
            pltpu.make_async_copy(
                sbuf.at[b], out_hbm.at[pl.ds(base, CHUNK)], ssem.at[b]
            ).wait()

        for b in range(NBUF):
            gather_start(b, b)

        for b in range(NBUF):
            gather_wait(b)
            scale(b)
            store_start(b, b)
            gather_start(b + NBUF, b)

        @pl.loop(NBUF, n_chunks - NBUF, step=NBUF)
        def _main(g0):
            for b in range(NBUF):
                g = g0 + b
                gather_wait(b)
                store_wait(b)
                scale(b)
                store_start(g, b)
                gather_start(g + NBUF, b)

        for b in range(NBUF):
            g = n_chunks - NBUF + b
            gather_wait(b)
            store_wait(b)
            scale(b)
            store_start(g, b)

        for b in range(NBUF):
            store_wait(b)

    return emb_kernel


def kernel(x, table):
    B = x.shape[0] * x.shape[1]
    idx = x.reshape(B).astype(jnp.int32)
    out = _build(B, table.shape[1], False)(idx, table)
    return out.reshape(x.shape[0], x.shape[1], table.shape[1])
